# baseline (speedup 1.0000x reference)
import jax
import jax.numpy as jnp
from jax.experimental import pallas as pl
from jax.experimental.pallas import tpu as pltpu
from jax.experimental.pallas import tpu_sc as plsc

_WINDOW = 128


def kernel(sequence, table):
    batch, hist = sequence.shape
    vocab, embed = table.shape
    n = batch * hist
    idx = sequence.reshape(1, n)

    mesh = plsc.VectorSubcoreMesh(core_axis_name="core", subcore_axis_name="subcore")

    @pl.kernel(
        out_type=jax.ShapeDtypeStruct((n, embed), table.dtype),
        mesh=mesh,
    )
    def _gather_kernel(table_hbm, idx_hbm, out_hbm):
        def body(idx_vmem, out_vmem):
            pltpu.sync_copy(table_hbm.at[idx_vmem.at[0]], out_vmem)

        pltpu.emit_pipeline(
            body,
            grid=(n // _WINDOW,),
            in_specs=[pl.BlockSpec((1, _WINDOW), index_map=lambda i: (0, i))],
            out_specs=[pl.BlockSpec((_WINDOW, embed), index_map=lambda i: (i, 0))],
            core_axis_name=("core", "subcore"),
            dimension_semantics=(pltpu.PARALLEL,),
        )(idx_hbm, out_hbm)

    out = _gather_kernel(table, idx)
    return out.reshape(batch, hist, embed)


# speedup vs baseline: 1.2490x; 1.2490x over previous
import jax
import jax.numpy as jnp
from jax import lax
from jax.experimental import pallas as pl
from jax.experimental.pallas import tpu as pltpu
from jax.experimental.pallas import tpu_sc as plsc

_W = 128
_NBUF = 4


def kernel(sequence, table):
    batch, hist = sequence.shape
    vocab, embed = table.shape
    n = batch * hist

    info = plsc.get_sparse_core_info()
    nc, ns = info.num_cores, info.num_subcores
    nw = nc * ns
    rows_per_worker = n // nw
    nwin = rows_per_worker // _W
    idx3 = sequence.reshape(nw, nwin, _W)

    mesh = plsc.VectorSubcoreMesh(core_axis_name="c", subcore_axis_name="s")

    @pl.kernel(
        out_type=jax.ShapeDtypeStruct((n, embed), table.dtype),
        mesh=mesh,
        scratch_types=[
            pltpu.VMEM((nwin, _W), jnp.int32),
            pltpu.VMEM((_NBUF, _W, embed), table.dtype),
        ]
        + [pltpu.SemaphoreType.DMA] * (2 * _NBUF),
    )
    def _gather_kernel(table_hbm, idx_hbm, out_hbm, idx_v, bufs, *sems):
        gsems = sems[:_NBUF]
        wsems = sems[_NBUF:]
        wid = lax.axis_index("s") * nc + lax.axis_index("c")
        base = wid * rows_per_worker

        pltpu.sync_copy(idx_hbm.at[wid], idx_v)

        for b in range(_NBUF):
            pltpu.make_async_copy(
                table_hbm.at[idx_v.at[b]], bufs.at[b], gsems[b]
            ).start()

        @pl.loop(0, nwin, step=_NBUF)
        def _(w0):
            for b in range(_NBUF):
                w = w0 + b
                pltpu.make_async_copy(
                    table_hbm.at[idx_v.at[w]], bufs.at[b], gsems[b]
                ).wait()
                dst = out_hbm.at[pl.ds(base + w * _W, _W)]
                pltpu.make_async_copy(bufs.at[b], dst, wsems[b]).start()

                nxt = w + _NBUF

                @pl.when(nxt < nwin)
                def _():
                    pltpu.make_async_copy(bufs.at[b], dst, wsems[b]).wait()
                    pltpu.make_async_copy(
                        table_hbm.at[idx_v.at[nxt]], bufs.at[b], gsems[b]
                    ).start()

        for b in range(_NBUF):
            pltpu.make_async_copy(
                bufs.at[b], out_hbm.at[pl.ds(base, _W)], wsems[b]
            ).wait()

    out = _gather_kernel(table, idx3)
    return out.reshape(batch, hist, embed)
